# baseline (speedup 1.0000x reference)
import jax
import jax.numpy as jnp
from jax.experimental import pallas as pl


def _graph_conv_plain(x, src, dst, W, b):
    n = x.shape[0]
    deg_out = jnp.zeros((n,), jnp.float32).at[src].add(1.0)
    deg_in = jnp.zeros((n,), jnp.float32).at[dst].add(1.0)
    norm_src = jnp.where(deg_out > 0, jax.lax.rsqrt(jnp.maximum(deg_out, 1e-12)), 0.0)
    norm_dst = jnp.where(deg_in > 0, jax.lax.rsqrt(jnp.maximum(deg_in, 1e-12)), 0.0)
    h = x * norm_src[:, None]
    agg = jnp.zeros((n, x.shape[1]), x.dtype).at[dst].add(h[src])
    agg = agg * norm_dst[:, None]
    return agg @ W + b


def kernel(x, edge_index_r0, edge_index_r1, edge_index_r2, W_r0, W_r1, W_r2, b_r0, b_r1, b_r2, a_type):
    outs = []
    for ei, W, b in ((edge_index_r0, W_r0, b_r0), (edge_index_r1, W_r1, b_r1), (edge_index_r2, W_r2, b_r2)):
        outs.append(_graph_conv_plain(x, ei[0], ei[1], W, b))
    h_stack = jnp.stack(outs, axis=0)
    mean_h = jnp.mean(h_stack, axis=0)
    scores = []
    for i in range(h_stack.shape[0]):
        cat = jnp.concatenate([h_stack[i], mean_h], axis=1)
        s = jax.nn.leaky_relu(jnp.sum(cat * a_type, axis=1, keepdims=True), negative_slope=0.2)
        scores.append(s)
    scores = jnp.stack(scores, axis=0)
    type_attn = jax.nn.softmax(scores, axis=0)
    h_final = jnp.sum(type_attn * h_stack, axis=0)
    return h_final


# speedup vs baseline: 1.9060x; 1.9060x over previous
import functools

import jax
import jax.numpy as jnp
from jax import lax
from jax.experimental import pallas as pl
from jax.experimental.pallas import tpu as pltpu
from jax.experimental.pallas import tpu_sc as plsc

N = 100000
D = 128
E = 500000
L = 16

_TILES_PER_ARRAY = 5
_EDGES_PER_TILE = E // _TILES_PER_ARRAY
_STAGE = 10000
_NCHUNK = _EDGES_PER_TILE // _STAGE


_W = 128
_CROWS = 512
_NPASS = 2
_NBATCH = _CROWS // 128


def _degree_body(edges, out, deg, stage, idx_buf, zbuf, spmem):
    c = lax.axis_index("c")
    s = lax.axis_index("s")
    a = c * 3 + s // _TILES_PER_ARRAY
    t = a // 2
    side = a % 2
    j = s % _TILES_PER_ARRAY
    al = s // _TILES_PER_ARRAY

    zeros16 = jnp.zeros((L,), jnp.float32)
    ones16 = jnp.ones((L,), jnp.float32)
    iota16 = lax.iota(jnp.int32, L)

    def idx_body(i, _):
        b = i // (128 // L)
        g = i % (128 // L)
        idx_buf[b, pl.ds(g * L, L)] = al * _CROWS + b * 128 + g * L + iota16
        return _

    lax.fori_loop(0, _NBATCH * (128 // L), idx_body, 0)

    def zero_z(i, _):
        zbuf[i // (_W // L), pl.ds((i % (_W // L)) * L, L)] = zeros16
        return _

    lax.fori_loop(0, 96 * (_W // L), zero_z, 0)

    for p in range(_NPASS):
        pltpu.sync_copy(zbuf, spmem.at[pl.ds(s * 96, 96)])
        plsc.subcore_barrier()
        for b in range(_NBATCH):
            pltpu.sync_copy(spmem.at[pl.ds(0, 128)], deg.at[pl.ds(b * 128, 128)])
        plsc.subcore_barrier()

        @pl.when(s < 15)
        def _():
            def chunk_body(k, _):
                off = (t * 2 + side) * E + j * _EDGES_PER_TILE + k * _STAGE
                pltpu.sync_copy(edges.at[pl.ds(off, _STAGE)], stage)

                def grp(g, _):
                    idx = stage[pl.ds(g * L, L)]
                    row = lax.shift_right_logical(idx, 7) - p * _CROWS
                    msk = (row >= 0) & (row < _CROWS)
                    rowc = jnp.minimum(jnp.maximum(row, 0), _CROWS - 1)
                    plsc.addupdate_scatter(
                        deg, [rowc, lax.bitwise_and(idx, 127)], ones16, mask=msk
                    )
                    return _

                return lax.fori_loop(0, _STAGE // L, grp, _)

            lax.fori_loop(0, _NCHUNK, chunk_body, 0)
            for b in range(_NBATCH):
                pltpu.sync_copy(
                    deg.at[pl.ds(b * 128, 128)], spmem.at[idx_buf.at[b]], add=True
                )

        plsc.subcore_barrier()

        @pl.when((j == 0) & (s < 15))
        def _():
            pltpu.sync_copy(
                spmem.at[pl.ds(al * _CROWS, _CROWS)],
                out.at[a, pl.ds(p * _CROWS, _CROWS)],
            )

        plsc.subcore_barrier()


def _compute_degrees(edges_stacked):
    mesh = plsc.VectorSubcoreMesh(
        core_axis_name="c", subcore_axis_name="s", num_cores=2, num_subcores=16
    )
    out = pl.kernel(
        _degree_body,
        out_type=jax.ShapeDtypeStruct((8, _NPASS * _CROWS, _W), jnp.float32),
        mesh=mesh,
        compiler_params=pltpu.CompilerParams(needs_layout_passes=False),
        scratch_types=[
            pltpu.VMEM((_CROWS, _W), jnp.float32),
            pltpu.VMEM((_STAGE,), jnp.int32),
            pltpu.VMEM((_NBATCH, 128), jnp.int32),
            pltpu.VMEM((96, _W), jnp.float32),
            pltpu.VMEM_SHARED((3 * _CROWS, _W), jnp.float32),
        ],
    )(edges_stacked)
    return out.reshape(8, _NPASS * _CROWS * _W)[:, :N]


_R = 12800
_NPASS_E = 4
_NPAD = _NPASS_E * 2 * _R
_SS = 2000
_FB = 128


def _agg_body(ec, h0, h1, h2, o0, o1, o2,
              ssrc, sdst, psrc, poff, rows, zbuf, sem, sem2, spmem):
    c = lax.axis_index("c")
    s = lax.axis_index("s")
    zeros16 = jnp.zeros((L,), jnp.float32)

    def zero_z(i, _):
        zbuf[i // 8, pl.ds((i % 8) * L, L)] = zeros16
        return _

    lax.fori_loop(0, 48 * 8, zero_z, 0)

    b0 = s * 16 - jnp.maximum(s - 10, 0)
    nblk = jnp.where(s < 10, 16, 15)

    for t, (h, out) in enumerate(((h0, o0), (h1, o1), (h2, o2))):
        for p in range(_NPASS_E):
            base = (p * 2 + c) * _R

            for k in range(16):
                pltpu.sync_copy(zbuf, spmem.at[pl.ds(s * 800 + k * 48, 48)])
            pltpu.sync_copy(
                zbuf.at[pl.ds(0, 32)], spmem.at[pl.ds(s * 800 + 768, 32)]
            )
            plsc.subcore_barrier()

            def fire():
                pltpu.async_copy(h.at[psrc.at[pl.ds(0, _FB)]], rows, sem).wait()
                descs = []
                for k in range(_FB // L):
                    idxv = poff[pl.ds(k * L, L)]
                    descs.append(
                        pltpu.async_copy(
                            rows.at[pl.ds(k * L, L)], spmem.at[idxv], sem2, add=True
                        )
                    )
                for dsc in descs:
                    dsc.wait()

            def block_body(k, cnt):
                eoff = (b0 + k) * _SS
                pltpu.sync_copy(ec.at[pl.ds(t * 2 * E + eoff, _SS)], ssrc)
                pltpu.sync_copy(ec.at[pl.ds((t * 2 + 1) * E + eoff, _SS)], sdst)

                def grp(g, cnt):
                    sv = ssrc[pl.ds(g * L, L)]
                    dv = sdst[pl.ds(g * L, L)]
                    off = dv - base
                    msk = (off >= 0) & (off < _R)
                    plsc.store_compressed(psrc.at[pl.ds(cnt, L)], sv, mask=msk)
                    plsc.store_compressed(poff.at[pl.ds(cnt, L)], off, mask=msk)
                    cnt = cnt + plsc.all_reduce_population_count(msk)[0]

                    @pl.when(cnt >= _FB)
                    def _():
                        fire()
                        psrc[pl.ds(0, L)] = psrc[pl.ds(_FB, L)]
                        poff[pl.ds(0, L)] = poff[pl.ds(_FB, L)]

                    return jnp.where(cnt >= _FB, cnt - _FB, cnt)

                return lax.fori_loop(0, _SS // L, grp, cnt)

            cnt = lax.fori_loop(0, nblk, block_body, jnp.int32(0))

            def padk(k, _):
                psrc[pl.ds(cnt + k * L, L)] = jnp.zeros((L,), jnp.int32)
                poff[pl.ds(cnt + k * L, L)] = jnp.full((L,), _R, jnp.int32)
                return _

            lax.fori_loop(0, _FB // L, padk, 0)
            fire()

            plsc.subcore_barrier()
            pltpu.sync_copy(
                spmem.at[pl.ds(s * 800, 800)],
                out.at[pl.ds(base + s * 800, 800)],
            )
            plsc.subcore_barrier()


def _edge_aggregate(edges_flat, h0, h1, h2):
    mesh = plsc.VectorSubcoreMesh(
        core_axis_name="c", subcore_axis_name="s", num_cores=2, num_subcores=16
    )
    outs = pl.kernel(
        _agg_body,
        out_type=[jax.ShapeDtypeStruct((_NPAD, D), jnp.float32)] * 3,
        mesh=mesh,
        compiler_params=pltpu.CompilerParams(needs_layout_passes=False),
        scratch_types=[
            pltpu.VMEM((_SS,), jnp.int32),
            pltpu.VMEM((_SS,), jnp.int32),
            pltpu.VMEM((256,), jnp.int32),
            pltpu.VMEM((256,), jnp.int32),
            pltpu.VMEM((_FB, D), jnp.float32),
            pltpu.VMEM((48, D), jnp.float32),
            pltpu.SemaphoreType.DMA,
            pltpu.SemaphoreType.DMA,
            pltpu.VMEM_SHARED((_R + 8, D), jnp.float32),
        ],
    )(edges_flat, h0, h1, h2)
    return [o[:N] for o in outs]


_BLK = 1000


def _norm_body(x_ref, degs_ref, h0_ref, h1_ref, h2_ref):
    xr = x_ref[...]
    for t, h_ref in enumerate((h0_ref, h1_ref, h2_ref)):
        d = degs_ref[:, 2 * t]
        norm = jnp.where(d > 0, lax.rsqrt(jnp.maximum(d, 1e-12)), 0.0)
        h_ref[...] = xr * norm[:, None]


def _apply_src_norm(x, degs):
    grid = N // _BLK
    return pl.pallas_call(
        _norm_body,
        grid=(grid,),
        in_specs=[
            pl.BlockSpec((_BLK, D), lambda i: (i, 0)),
            pl.BlockSpec((_BLK, 8), lambda i: (i, 0)),
        ],
        out_specs=[pl.BlockSpec((_BLK, D), lambda i: (i, 0))] * 3,
        out_shape=[jax.ShapeDtypeStruct((N, D), jnp.float32)] * 3,
    )(x, degs)


def _combine_body(a0_ref, a1_ref, a2_ref, degs_ref, w_ref, bb_ref, at_ref, out_ref):
    os = []
    for t, a_ref in enumerate((a0_ref, a1_ref, a2_ref)):
        d = degs_ref[:, 2 * t + 1]
        norm = jnp.where(d > 0, lax.rsqrt(jnp.maximum(d, 1e-12)), 0.0)
        g = a_ref[...] * norm[:, None]
        o = jnp.dot(g, w_ref[t], preferred_element_type=jnp.float32) + bb_ref[t, 0][None, :]
        os.append(o)
    mean = (os[0] + os[1] + os[2]) * (1.0 / 3.0)
    a1v = at_ref[0, 0][None, :]
    a2v = at_ref[0, 1][None, :]
    mscore = jnp.sum(mean * a2v, axis=1)
    ss = []
    for t in range(3):
        sv = jnp.sum(os[t] * a1v, axis=1) + mscore
        ss.append(jnp.where(sv > 0, sv, 0.2 * sv))
    m = jnp.maximum(jnp.maximum(ss[0], ss[1]), ss[2])
    es = [jnp.exp(sv - m) for sv in ss]
    denom = es[0] + es[1] + es[2]
    acc = es[0][:, None] * os[0] + es[1][:, None] * os[1] + es[2][:, None] * os[2]
    out_ref[...] = acc / denom[:, None]


def _combine(agg0, agg1, agg2, degs, Ws, bs, a_type):
    grid = N // _BLK
    return pl.pallas_call(
        _combine_body,
        grid=(grid,),
        in_specs=[
            pl.BlockSpec((_BLK, D), lambda i: (i, 0)),
            pl.BlockSpec((_BLK, D), lambda i: (i, 0)),
            pl.BlockSpec((_BLK, D), lambda i: (i, 0)),
            pl.BlockSpec((_BLK, 8), lambda i: (i, 0)),
            pl.BlockSpec((3, D, D), lambda i: (0, 0, 0)),
            pl.BlockSpec((3, 1, D), lambda i: (0, 0, 0)),
            pl.BlockSpec((1, 2, D), lambda i: (0, 0, 0)),
        ],
        out_specs=pl.BlockSpec((_BLK, D), lambda i: (i, 0)),
        out_shape=jax.ShapeDtypeStruct((N, D), jnp.float32),
    )(agg0, agg1, agg2, degs, Ws, bs, a_type)


def kernel(x, edge_index_r0, edge_index_r1, edge_index_r2, W_r0, W_r1, W_r2, b_r0, b_r1, b_r2, a_type):
    edges = jnp.stack([edge_index_r0, edge_index_r1, edge_index_r2]).reshape(-1)
    degs = _compute_degrees(edges).T
    hs = _apply_src_norm(x, degs)

    aggs = _edge_aggregate(edges, hs[0], hs[1], hs[2])

    Ws = jnp.stack([W_r0, W_r1, W_r2])
    bs = jnp.stack([b_r0, b_r1, b_r2]).reshape(3, 1, D)
    at = a_type.reshape(1, 2, D)
    return _combine(aggs[0], aggs[1], aggs[2], degs, Ws, bs, at)


# speedup vs baseline: 1.9451x; 1.0205x over previous
import functools

import jax
import jax.numpy as jnp
from jax import lax
from jax.experimental import pallas as pl
from jax.experimental.pallas import tpu as pltpu
from jax.experimental.pallas import tpu_sc as plsc

N = 100000
D = 128
E = 500000
L = 16

_TILES_PER_ARRAY = 5
_EDGES_PER_TILE = E // _TILES_PER_ARRAY
_STAGE = 10000
_NCHUNK = _EDGES_PER_TILE // _STAGE


_W = 128
_CROWS = 512
_NPASS = 2
_NBATCH = _CROWS // 128


def _degree_body(edges, out, deg, stage, idx_buf, zbuf, spmem):
    c = lax.axis_index("c")
    s = lax.axis_index("s")
    a = c * 3 + s // _TILES_PER_ARRAY
    t = a // 2
    side = a % 2
    j = s % _TILES_PER_ARRAY
    al = s // _TILES_PER_ARRAY

    zeros16 = jnp.zeros((L,), jnp.float32)
    ones16 = jnp.ones((L,), jnp.float32)
    iota16 = lax.iota(jnp.int32, L)

    def idx_body(i, _):
        b = i // (128 // L)
        g = i % (128 // L)
        idx_buf[b, pl.ds(g * L, L)] = al * _CROWS + b * 128 + g * L + iota16
        return _

    lax.fori_loop(0, _NBATCH * (128 // L), idx_body, 0)

    def zero_z(i, _):
        zbuf[i // (_W // L), pl.ds((i % (_W // L)) * L, L)] = zeros16
        return _

    lax.fori_loop(0, 96 * (_W // L), zero_z, 0)

    for p in range(_NPASS):
        pltpu.sync_copy(zbuf, spmem.at[pl.ds(s * 96, 96)])
        plsc.subcore_barrier()
        for b in range(_NBATCH):
            pltpu.sync_copy(spmem.at[pl.ds(0, 128)], deg.at[pl.ds(b * 128, 128)])
        plsc.subcore_barrier()

        @pl.when(s < 15)
        def _():
            def chunk_body(k, _):
                off = (t * 2 + side) * E + j * _EDGES_PER_TILE + k * _STAGE
                pltpu.sync_copy(edges.at[pl.ds(off, _STAGE)], stage)

                def grp(g, _):
                    idx = stage[pl.ds(g * L, L)]
                    row = lax.shift_right_logical(idx, 7) - p * _CROWS
                    msk = (row >= 0) & (row < _CROWS)
                    rowc = jnp.minimum(jnp.maximum(row, 0), _CROWS - 1)
                    plsc.addupdate_scatter(
                        deg, [rowc, lax.bitwise_and(idx, 127)], ones16, mask=msk
                    )
                    return _

                return lax.fori_loop(0, _STAGE // L, grp, _)

            lax.fori_loop(0, _NCHUNK, chunk_body, 0)
            for b in range(_NBATCH):
                pltpu.sync_copy(
                    deg.at[pl.ds(b * 128, 128)], spmem.at[idx_buf.at[b]], add=True
                )

        plsc.subcore_barrier()

        @pl.when((j == 0) & (s < 15))
        def _():
            pltpu.sync_copy(
                spmem.at[pl.ds(al * _CROWS, _CROWS)],
                out.at[a, pl.ds(p * _CROWS, _CROWS)],
            )

        plsc.subcore_barrier()


def _compute_degrees(edges_stacked):
    mesh = plsc.VectorSubcoreMesh(
        core_axis_name="c", subcore_axis_name="s", num_cores=2, num_subcores=16
    )
    out = pl.kernel(
        _degree_body,
        out_type=jax.ShapeDtypeStruct((8, _NPASS * _CROWS, _W), jnp.float32),
        mesh=mesh,
        compiler_params=pltpu.CompilerParams(needs_layout_passes=False),
        scratch_types=[
            pltpu.VMEM((_CROWS, _W), jnp.float32),
            pltpu.VMEM((_STAGE,), jnp.int32),
            pltpu.VMEM((_NBATCH, 128), jnp.int32),
            pltpu.VMEM((96, _W), jnp.float32),
            pltpu.VMEM_SHARED((3 * _CROWS, _W), jnp.float32),
        ],
    )(edges_stacked)
    return out.reshape(8, _NPASS * _CROWS * _W)[:, :N]


_R = 12800
_NPASS_E = 4
_NPAD = _NPASS_E * 2 * _R
_SS = 2000
_FB = 128


def _agg_body(ec, h0, h1, h2, o0, o1, o2,
              ssrc, sdst, psrc, poff, rows, zbuf, sem, sem2, spmem):
    c = lax.axis_index("c")
    s = lax.axis_index("s")
    zeros16 = jnp.zeros((L,), jnp.float32)

    def zero_z(i, _):
        zbuf[i // 8, pl.ds((i % 8) * L, L)] = zeros16
        return _

    lax.fori_loop(0, 48 * 8, zero_z, 0)

    b0 = s * 16 - jnp.maximum(s - 10, 0)
    nblk = jnp.where(s < 10, 16, 15)

    for t, (h, out) in enumerate(((h0, o0), (h1, o1), (h2, o2))):
        for p in range(_NPASS_E):
            base = (p * 2 + c) * _R

            for k in range(16):
                pltpu.sync_copy(zbuf, spmem.at[pl.ds(s * 800 + k * 48, 48)])
            pltpu.sync_copy(
                zbuf.at[pl.ds(0, 32)], spmem.at[pl.ds(s * 800 + 768, 32)]
            )
            plsc.subcore_barrier()

            def fire():
                pltpu.async_copy(h.at[psrc.at[pl.ds(0, _FB)]], rows, sem).wait()
                descs = []
                for k in range(_FB // L):
                    idxv = poff[pl.ds(k * L, L)]
                    descs.append(
                        pltpu.async_copy(
                            rows.at[pl.ds(k * L, L)], spmem.at[idxv], sem2, add=True
                        )
                    )
                for dsc in descs:
                    dsc.wait()

            def block_body(k, cnt):
                eoff = (b0 + k) * _SS
                pltpu.sync_copy(ec.at[pl.ds(t * 2 * E + eoff, _SS)], ssrc)
                pltpu.sync_copy(ec.at[pl.ds((t * 2 + 1) * E + eoff, _SS)], sdst)

                def grp(g, cnt):
                    sv = ssrc[pl.ds(g * L, L)]
                    dv = sdst[pl.ds(g * L, L)]
                    off = dv - base
                    msk = (off >= 0) & (off < _R)
                    plsc.store_compressed(psrc.at[pl.ds(cnt, L)], sv, mask=msk)
                    plsc.store_compressed(poff.at[pl.ds(cnt, L)], off, mask=msk)
                    cnt = cnt + plsc.all_reduce_population_count(msk)[0]

                    @pl.when(cnt >= _FB)
                    def _():
                        fire()
                        psrc[pl.ds(0, L)] = psrc[pl.ds(_FB, L)]
                        poff[pl.ds(0, L)] = poff[pl.ds(_FB, L)]

                    return jnp.where(cnt >= _FB, cnt - _FB, cnt)

                return lax.fori_loop(0, _SS // L, grp, cnt)

            cnt = lax.fori_loop(0, nblk, block_body, jnp.int32(0))

            def padk(k, carry):
                psrc[pl.ds(cnt + k * L, L)] = jnp.zeros((L,), jnp.int32)
                poff[pl.ds(cnt + k * L, L)] = jnp.full((L,), _R, jnp.int32)
                return carry

            lax.fori_loop(0, _FB // L, padk, 0)
            fire()

            plsc.subcore_barrier()
            pltpu.sync_copy(
                spmem.at[pl.ds(s * 800, 800)],
                out.at[pl.ds(base + s * 800, 800)],
            )
            plsc.subcore_barrier()


def _edge_aggregate(edges_flat, h0, h1, h2):
    mesh = plsc.VectorSubcoreMesh(
        core_axis_name="c", subcore_axis_name="s", num_cores=2, num_subcores=16
    )
    outs = pl.kernel(
        _agg_body,
        out_type=[jax.ShapeDtypeStruct((_NPAD, D), jnp.float32)] * 3,
        mesh=mesh,
        compiler_params=pltpu.CompilerParams(needs_layout_passes=False),
        scratch_types=[
            pltpu.VMEM((_SS,), jnp.int32),
            pltpu.VMEM((_SS,), jnp.int32),
            pltpu.VMEM((256,), jnp.int32),
            pltpu.VMEM((256,), jnp.int32),
            pltpu.VMEM((_FB, D), jnp.float32),
            pltpu.VMEM((48, D), jnp.float32),
            pltpu.SemaphoreType.DMA,
            pltpu.SemaphoreType.DMA,
            pltpu.VMEM_SHARED((_R + 8, D), jnp.float32),
        ],
    )(edges_flat, h0, h1, h2)
    return outs


_BLK = 1000


def _norm_body(x_ref, degs_ref, h0_ref, h1_ref, h2_ref):
    xr = x_ref[...]
    for t, h_ref in enumerate((h0_ref, h1_ref, h2_ref)):
        d = degs_ref[:, 2 * t]
        norm = jnp.where(d > 0, lax.rsqrt(jnp.maximum(d, 1e-12)), 0.0)
        h_ref[...] = xr * norm[:, None]


def _apply_src_norm(x, degs):
    grid = N // _BLK
    return pl.pallas_call(
        _norm_body,
        grid=(grid,),
        in_specs=[
            pl.BlockSpec((_BLK, D), lambda i: (i, 0)),
            pl.BlockSpec((_BLK, 8), lambda i: (i, 0)),
        ],
        out_specs=[pl.BlockSpec((_BLK, D), lambda i: (i, 0))] * 3,
        out_shape=[jax.ShapeDtypeStruct((N, D), jnp.float32)] * 3,
    )(x, degs)


def _combine_body(a0_ref, a1_ref, a2_ref, degs_ref, w_ref, bb_ref, at_ref, out_ref):
    os = []
    for t, a_ref in enumerate((a0_ref, a1_ref, a2_ref)):
        d = degs_ref[:, 2 * t + 1]
        norm = jnp.where(d > 0, lax.rsqrt(jnp.maximum(d, 1e-12)), 0.0)
        g = a_ref[...] * norm[:, None]
        o = jnp.dot(g, w_ref[t], preferred_element_type=jnp.float32) + bb_ref[t, 0][None, :]
        os.append(o)
    mean = (os[0] + os[1] + os[2]) * (1.0 / 3.0)
    a1v = at_ref[0, 0][None, :]
    a2v = at_ref[0, 1][None, :]
    mscore = jnp.sum(mean * a2v, axis=1)
    ss = []
    for t in range(3):
        sv = jnp.sum(os[t] * a1v, axis=1) + mscore
        ss.append(jnp.where(sv > 0, sv, 0.2 * sv))
    m = jnp.maximum(jnp.maximum(ss[0], ss[1]), ss[2])
    es = [jnp.exp(sv - m) for sv in ss]
    denom = es[0] + es[1] + es[2]
    acc = es[0][:, None] * os[0] + es[1][:, None] * os[1] + es[2][:, None] * os[2]
    out_ref[...] = acc / denom[:, None]


def _combine(agg0, agg1, agg2, degs, Ws, bs, a_type):
    grid = N // _BLK
    return pl.pallas_call(
        _combine_body,
        grid=(grid,),
        in_specs=[
            pl.BlockSpec((_BLK, D), lambda i: (i, 0)),
            pl.BlockSpec((_BLK, D), lambda i: (i, 0)),
            pl.BlockSpec((_BLK, D), lambda i: (i, 0)),
            pl.BlockSpec((_BLK, 8), lambda i: (i, 0)),
            pl.BlockSpec((3, D, D), lambda i: (0, 0, 0)),
            pl.BlockSpec((3, 1, D), lambda i: (0, 0, 0)),
            pl.BlockSpec((1, 2, D), lambda i: (0, 0, 0)),
        ],
        out_specs=pl.BlockSpec((_BLK, D), lambda i: (i, 0)),
        out_shape=jax.ShapeDtypeStruct((N, D), jnp.float32),
    )(agg0, agg1, agg2, degs, Ws, bs, a_type)


def kernel(x, edge_index_r0, edge_index_r1, edge_index_r2, W_r0, W_r1, W_r2, b_r0, b_r1, b_r2, a_type):
    edges = jnp.stack([edge_index_r0, edge_index_r1, edge_index_r2]).reshape(-1)
    degs = _compute_degrees(edges).T
    hs = _apply_src_norm(x, degs)

    aggs = _edge_aggregate(edges, hs[0], hs[1], hs[2])

    Ws = jnp.stack([W_r0, W_r1, W_r2])
    bs = jnp.stack([b_r0, b_r1, b_r2]).reshape(3, 1, D)
    at = a_type.reshape(1, 2, D)
    return _combine(aggs[0], aggs[1], aggs[2], degs, Ws, bs, at)
